# baseline (speedup 1.0000x reference)
import functools

import jax
import jax.numpy as jnp
from jax import lax
from jax.experimental import pallas as pl
from jax.experimental.pallas import tpu as pltpu
from jax.experimental.pallas import tpu_sc as plsc

N = 12
E = 768
K = 125
S = 8
KS = 5
NW = 16
EPW = E // NW
KP = 128
CP = N * KP
GRP = EPW * S // 16


def _sc_c_body(ei_hbm, attr_hbm, out_hbm, src_v, dst_v, attr_v, cl):
    wid = lax.axis_index("s") * 2 + lax.axis_index("c")
    e0 = wid * EPW
    pl.when(wid < NW)(lambda: _sc_c_work(
        ei_hbm, attr_hbm, out_hbm, src_v, dst_v, attr_v, cl, wid, e0))


def _sc_c_work(ei_hbm, attr_hbm, out_hbm, src_v, dst_v, attr_v, cl, wid, e0):
    pltpu.sync_copy(ei_hbm.at[0, pl.ds(e0, EPW)], src_v)
    pltpu.sync_copy(ei_hbm.at[1, pl.ds(e0, EPW)], dst_v)
    pltpu.sync_copy(attr_hbm.at[pl.ds(e0 * 3, EPW * 3)], attr_v)
    for z in range(N * CP // 16):
        cl[pl.ds(z * 16, 16)] = jnp.zeros((16,), jnp.float32)

    lanes = lax.broadcasted_iota(jnp.int32, (16,), 0)
    for g in range(GRP):
        p = g * 16 + lanes
        e = p >> 3
        s = p & 7
        srcp = plsc.load_gather(src_v, [e])
        dstp = plsc.load_gather(dst_v, [e])
        b = jnp.ones((16,), jnp.float32)
        wi = jnp.zeros((16,), jnp.int32)
        stride = 1
        for d in range(3):
            v = plsc.load_gather(attr_v, [e * 3 + d]) * (KS - 1.0)
            boti = jnp.minimum(v.astype(jnp.int32), KS - 2)
            frac = v - boti.astype(jnp.float32)
            bit = (s >> d) & 1
            b = b * jnp.where(bit == 1, frac, 1.0 - frac)
            wi = wi + (boti + bit) * stride
            stride *= KS
        idx = dstp * CP + srcp * KP + wi
        for j in range(16):
            plsc.addupdate_scatter(cl, [idx], b, mask=lanes == j)
    pltpu.sync_copy(cl, out_hbm.at[wid])


def _sc_c(edge_index, attr_flat):
    mesh = plsc.VectorSubcoreMesh(core_axis_name="c", subcore_axis_name="s")
    kfn = functools.partial(
        pl.kernel,
        mesh=mesh,
        out_type=jax.ShapeDtypeStruct((NW, N * CP), jnp.float32),
        scratch_types=[
            pltpu.VMEM((EPW,), jnp.int32),
            pltpu.VMEM((EPW,), jnp.int32),
            pltpu.VMEM((EPW * 3,), jnp.float32),
            pltpu.VMEM((N * CP,), jnp.float32),
        ],
        compiler_params=pltpu.CompilerParams(use_tc_tiling_on_sc=False,
                                             needs_layout_passes=False),
    )(_sc_c_body)
    return kfn(edge_index, attr_flat)


def _elu(a):
    return jnp.where(a > 0.0, a, jnp.exp(jnp.minimum(a, 0.0)) - 1.0)


def _tc1_body(x_ref, w1_ref, root1_ref, b1_ref, xw_ref, xr_ref):
    xw3 = lax.dot_general(x_ref[...], w1_ref[...], (((1,), (1,)), ((), ())),
                          preferred_element_type=jnp.float32)
    xw_ref[...] = jnp.pad(xw3, ((0, 0), (0, KP - K), (0, 0)))
    xr_ref[...] = jnp.dot(x_ref[...], root1_ref[...],
                          preferred_element_type=jnp.float32) + b1_ref[...]


def _tc1(x, w1, root1, b1):
    return pl.pallas_call(
        _tc1_body,
        out_shape=(
            jax.ShapeDtypeStruct((N, KP, 32), jnp.float32),
            jax.ShapeDtypeStruct((N, 32), jnp.float32),
        ),
    )(x, w1, root1, b1)


def _csum_cnt(call_ref, dst_ref):
    c = call_ref[pl.ds(0, N), :]
    for w in range(1, NW):
        c = c + call_ref[pl.ds(w * N, N), :]
    onehot = jnp.where(
        lax.broadcasted_iota(jnp.int32, (N, E), 0) == dst_ref[1:2, :],
        1.0, 0.0)
    cnt = jnp.maximum(jnp.sum(onehot, axis=1, keepdims=True), 1.0)
    return c, cnt


def _tcm_body(call_ref, dst_ref, xw1_ref, xr_ref, w2_ref, root2_ref, b2_ref,
              fc1w_ref, fc1b_ref, fc2w_ref, fc2b_ref, out_ref):
    c, cnt = _csum_cnt(call_ref, dst_ref)
    agg1 = jnp.dot(c, xw1_ref[...],
                   preferred_element_type=jnp.float32) / cnt
    h1 = _elu(agg1 + xr_ref[...])
    sub = lax.broadcasted_iota(jnp.int32, (32, 32 * KP), 0)
    lane = lax.broadcasted_iota(jnp.int32, (32, 32 * KP), 1)
    onehot = jnp.where(sub == lane // KP, 1.0, 0.0)
    hrows = jnp.dot(h1, onehot, preferred_element_type=jnp.float32)
    zrows = jnp.zeros((KP - K, 64), jnp.float32)
    agg2 = jnp.zeros((N, 64), jnp.float32)
    for i in range(32):
        mi = hrows[0:1, i * KP:(i + 1) * KP] * c[:, 0:KP]
        for n in range(1, N):
            mi = mi + (hrows[n:n + 1, i * KP:(i + 1) * KP]
                       * c[:, n * KP:(n + 1) * KP])
        w2i = jnp.concatenate([w2_ref[:, i, :], zrows], axis=0)
        agg2 = agg2 + jnp.dot(mi, w2i, preferred_element_type=jnp.float32)
    agg2 = agg2 / cnt
    h2 = _elu(agg2 + jnp.dot(h1, root2_ref[...],
                             preferred_element_type=jnp.float32)
              + b2_ref[...])
    y = fc1b_ref[...]
    for n in range(N):
        y = y + jnp.dot(h2[n:n + 1, :], fc1w_ref[pl.ds(n * 64, 64), :],
                        preferred_element_type=jnp.float32)
    z = jnp.dot(y, fc2w_ref[...],
                preferred_element_type=jnp.float32) + fc2b_ref[...]
    m = jnp.max(z, axis=1, keepdims=True)
    out_ref[...] = z - (m + jnp.log(jnp.sum(jnp.exp(z - m), axis=1,
                                            keepdims=True)))


def _tcm(call, dst, xw1flat, xr, w2s, root2, b2, fc1_w, fc1_b, fc2_w, fc2_b):
    return pl.pallas_call(
        _tcm_body,
        out_shape=jax.ShapeDtypeStruct((1, 2), jnp.float32),
    )(call, dst, xw1flat, xr, w2s, root2, b2, fc1_w, fc1_b, fc2_w, fc2_b)


def kernel(x, edge_index, edge_attr, W1, root1, b1, W2, root2, b2,
           fc1_w, fc1_b, fc2_w, fc2_b):
    call = _sc_c(edge_index, edge_attr.reshape(-1))
    xw1, xr = _tc1(x, W1, root1, b1.reshape(1, 32))
    return _tcm(call.reshape(NW * N, CP), edge_index,
                xw1.reshape(N * KP, 32), xr, W2, root2, b2.reshape(1, 64),
                fc1_w, fc1_b.reshape(1, 128), fc2_w, fc2_b.reshape(1, 2))


# speedup vs baseline: 1.0004x; 1.0004x over previous
import functools

import jax
import jax.numpy as jnp
from jax import lax
from jax.experimental import pallas as pl
from jax.experimental.pallas import tpu as pltpu
from jax.experimental.pallas import tpu_sc as plsc

N = 12
E = 768
K = 125
S = 8
KS = 5
NW = 16
EPW = E // NW
KP = 128
CP = N * KP
GRP = EPW * S // 16


def _sc_c_body(ei_hbm, attr_hbm, out_hbm, src_v, dst_v, attr_v, cl):
    wid = lax.axis_index("s") * 2 + lax.axis_index("c")
    e0 = wid * EPW
    pl.when(wid < NW)(lambda: _sc_c_work(
        ei_hbm, attr_hbm, out_hbm, src_v, dst_v, attr_v, cl, wid, e0))


def _sc_c_work(ei_hbm, attr_hbm, out_hbm, src_v, dst_v, attr_v, cl, wid, e0):
    pltpu.sync_copy(ei_hbm.at[0, pl.ds(e0, EPW)], src_v)
    pltpu.sync_copy(ei_hbm.at[1, pl.ds(e0, EPW)], dst_v)
    pltpu.sync_copy(attr_hbm.at[pl.ds(e0, EPW), :], attr_v)
    for r in range(N):
        for z in range(CP // 16):
            cl[r, pl.ds(z * 16, 16)] = jnp.zeros((16,), jnp.float32)

    lanes = lax.broadcasted_iota(jnp.int32, (16,), 0)
    for g in range(GRP):
        p = g * 16 + lanes
        e = p >> 3
        s = p & 7
        srcp = plsc.load_gather(src_v, [e])
        dstp = plsc.load_gather(dst_v, [e])
        b = jnp.ones((16,), jnp.float32)
        wi = jnp.zeros((16,), jnp.int32)
        stride = 1
        for d in range(3):
            v = plsc.load_gather(attr_v, [e, jnp.full((16,), d, jnp.int32)])
            v = v * (KS - 1.0)
            boti = jnp.minimum(v.astype(jnp.int32), KS - 2)
            frac = v - boti.astype(jnp.float32)
            bit = (s >> d) & 1
            b = b * jnp.where(bit == 1, frac, 1.0 - frac)
            wi = wi + (boti + bit) * stride
            stride *= KS
        col = srcp * KP + wi
        for j in range(16):
            plsc.addupdate_scatter(cl, [dstp, col], b, mask=lanes == j)
    pltpu.sync_copy(cl, out_hbm.at[pl.ds(wid * N, N), :])


def _sc_c(edge_index, edge_attr):
    mesh = plsc.VectorSubcoreMesh(core_axis_name="c", subcore_axis_name="s")
    kfn = functools.partial(
        pl.kernel,
        mesh=mesh,
        out_type=jax.ShapeDtypeStruct((NW * N, CP), jnp.float32),
        scratch_types=[
            pltpu.VMEM((EPW,), jnp.int32),
            pltpu.VMEM((EPW,), jnp.int32),
            pltpu.VMEM((EPW, 3), jnp.float32),
            pltpu.VMEM((N, CP), jnp.float32),
        ],
        compiler_params=pltpu.CompilerParams(use_tc_tiling_on_sc=False,
                                             needs_layout_passes=False),
    )(_sc_c_body)
    return kfn(edge_index, edge_attr)


def _elu(a):
    return jnp.where(a > 0.0, a, jnp.exp(jnp.minimum(a, 0.0)) - 1.0)


def _tc1_body(x_ref, w1_ref, root1_ref, b1_ref, xw_ref, xr_ref):
    xw3 = lax.dot_general(x_ref[...], w1_ref[...], (((1,), (1,)), ((), ())),
                          preferred_element_type=jnp.float32)
    zpad = jnp.zeros((KP - K, 32), jnp.float32)
    for n in range(N):
        xw_ref[pl.ds(n * KP, K), :] = xw3[n]
        xw_ref[pl.ds(n * KP + K, KP - K), :] = zpad
    xr_ref[...] = jnp.dot(x_ref[...], root1_ref[...],
                          preferred_element_type=jnp.float32) + b1_ref[...]


def _tc1(x, w1, root1, b1):
    return pl.pallas_call(
        _tc1_body,
        out_shape=(
            jax.ShapeDtypeStruct((N * KP, 32), jnp.float32),
            jax.ShapeDtypeStruct((N, 32), jnp.float32),
        ),
    )(x, w1, root1, b1)


def _csum_cnt(call_ref, dst_ref):
    c = call_ref[pl.ds(0, N), :]
    for w in range(1, NW):
        c = c + call_ref[pl.ds(w * N, N), :]
    onehot = jnp.where(
        lax.broadcasted_iota(jnp.int32, (N, E), 0) == dst_ref[1:2, :],
        1.0, 0.0)
    cnt = jnp.maximum(jnp.sum(onehot, axis=1, keepdims=True), 1.0)
    return c, cnt


def _tcm_body(call_ref, dst_ref, xw1_ref, xr_ref, w2_ref, root2_ref, b2_ref,
              fc1w_ref, fc1b_ref, fc2w_ref, fc2b_ref, out_ref):
    c, cnt = _csum_cnt(call_ref, dst_ref)
    agg1 = jnp.dot(c, xw1_ref[...],
                   preferred_element_type=jnp.float32) / cnt
    h1 = _elu(agg1 + xr_ref[...])
    sub = lax.broadcasted_iota(jnp.int32, (32, 32 * KP), 0)
    lane = lax.broadcasted_iota(jnp.int32, (32, 32 * KP), 1)
    onehot = jnp.where(sub == lane // KP, 1.0, 0.0)
    hrows = jnp.dot(h1, onehot, preferred_element_type=jnp.float32)
    zrows = jnp.zeros((KP - K, 64), jnp.float32)
    agg2 = jnp.zeros((N, 64), jnp.float32)
    for i in range(32):
        mi = hrows[0:1, i * KP:(i + 1) * KP] * c[:, 0:KP]
        for n in range(1, N):
            mi = mi + (hrows[n:n + 1, i * KP:(i + 1) * KP]
                       * c[:, n * KP:(n + 1) * KP])
        w2i = jnp.concatenate([w2_ref[:, i, :], zrows], axis=0)
        agg2 = agg2 + jnp.dot(mi, w2i, preferred_element_type=jnp.float32)
    agg2 = agg2 / cnt
    h2 = _elu(agg2 + jnp.dot(h1, root2_ref[...],
                             preferred_element_type=jnp.float32)
              + b2_ref[...])
    y = fc1b_ref[...]
    for n in range(N):
        y = y + jnp.dot(h2[n:n + 1, :], fc1w_ref[pl.ds(n * 64, 64), :],
                        preferred_element_type=jnp.float32)
    z = jnp.dot(y, fc2w_ref[...],
                preferred_element_type=jnp.float32) + fc2b_ref[...]
    m = jnp.max(z, axis=1, keepdims=True)
    out_ref[...] = z - (m + jnp.log(jnp.sum(jnp.exp(z - m), axis=1,
                                            keepdims=True)))


def _tcm(call, dst, xw1flat, xr, w2s, root2, b2, fc1_w, fc1_b, fc2_w, fc2_b):
    return pl.pallas_call(
        _tcm_body,
        out_shape=jax.ShapeDtypeStruct((1, 2), jnp.float32),
    )(call, dst, xw1flat, xr, w2s, root2, b2, fc1_w, fc1_b, fc2_w, fc2_b)


def kernel(x, edge_index, edge_attr, W1, root1, b1, W2, root2, b2,
           fc1_w, fc1_b, fc2_w, fc2_b):
    call = _sc_c(edge_index, edge_attr)
    xw1, xr = _tc1(x, W1, root1, b1.reshape(1, 32))
    return _tcm(call, edge_index, xw1, xr, W2, root2, b2.reshape(1, 64),
                fc1_w, fc1_b.reshape(1, 128), fc2_w, fc2_b.reshape(1, 2))
